# Initial kernel scaffold; baseline (speedup 1.0000x reference)
import jax
import jax.numpy as jnp
from jax import lax
from jax.experimental import pallas as pl
from jax.experimental.pallas import tpu as pltpu
from jax.experimental.pallas import tpu_sc as plsc

N = 10000
D = 256
OUT = 128
E = 160000

NPAD = 10240
ROWS_PER_TILE = NPAD // 16
NC, NS = 2, 16
NW = NC * NS
CH = 128
NCH = 40
EPAD = NW * NCH * CH
NL = 5
TOT_CH = NL * NCH

_MESH = plsc.VectorSubcoreMesh(
    core_axis_name="c", subcore_axis_name="s", num_cores=NC, num_subcores=NS)


def _hist_body(s_hbm, r_hbm, z_hbm, hs_out, hr_out, idx_v, ones_v, hs_sh, hr_sh):
    c = lax.axis_index("c")
    s = lax.axis_index("s")
    wid = c * NS + s
    one = jnp.ones((16,), jnp.float32)
    for i in range(8):
        ones_v[pl.ds(i * 16, 16)] = one
    for k in range(NL):
        pltpu.sync_copy(z_hbm, hs_sh.at[pl.ds(k * NPAD + s * ROWS_PER_TILE,
                                              ROWS_PER_TILE)])
        pltpu.sync_copy(z_hbm, hr_sh.at[pl.ds(k * NPAD + s * ROWS_PER_TILE,
                                              ROWS_PER_TILE)])
    plsc.subcore_barrier()

    pltpu.sync_copy(s_hbm.at[wid], idx_v)

    def sbody(j, carry):
        pltpu.sync_copy(ones_v, hs_sh.at[idx_v.at[j]], add=True)
        return carry

    lax.fori_loop(0, TOT_CH, sbody, 0)

    pltpu.sync_copy(r_hbm.at[wid], idx_v)

    def rbody(j, carry):
        pltpu.sync_copy(ones_v, hr_sh.at[idx_v.at[j]], add=True)
        return carry

    lax.fori_loop(0, TOT_CH, rbody, 0)
    plsc.subcore_barrier()
    for k in range(NL):
        sl = pl.ds(k * NPAD + s * ROWS_PER_TILE, ROWS_PER_TILE)
        osl = pl.ds(s * ROWS_PER_TILE, ROWS_PER_TILE)
        pltpu.sync_copy(hs_sh.at[sl], hs_out.at[k, c, osl])
        pltpu.sync_copy(hr_sh.at[sl], hr_out.at[k, c, osl])


def _hist_call(s_w, r_w, zeros1):
    f = pl.kernel(
        _hist_body,
        out_type=(jax.ShapeDtypeStruct((NL, NC, NPAD), jnp.float32),
                  jax.ShapeDtypeStruct((NL, NC, NPAD), jnp.float32)),
        mesh=_MESH,
        scratch_types=[
            pltpu.VMEM((TOT_CH, CH), jnp.int32),
            pltpu.VMEM((CH,), jnp.float32),
            pltpu.VMEM_SHARED((NL * NPAD,), jnp.float32),
            pltpu.VMEM_SHARED((NL * NPAD,), jnp.float32),
        ],
    )
    return f(s_w, r_w, zeros1)


def _scat_body(hs_hbm, s_hbm, r_hbm, z_hbm, parts_out,
               idx_s, idx_r, rows_v, acc_sh, sem0, sem1):
    c = lax.axis_index("c")
    s = lax.axis_index("s")
    wid = c * NS + s
    pltpu.sync_copy(s_hbm.at[wid], idx_s)
    pltpu.sync_copy(r_hbm.at[wid], idx_r)
    tsl = pl.ds(s * ROWS_PER_TILE, ROWS_PER_TILE)
    for l in range(NL):
        @pl.when(c == 0)
        def _():
            pltpu.sync_copy(hs_hbm.at[pl.ds(l * NPAD + s * ROWS_PER_TILE,
                                            ROWS_PER_TILE)], acc_sh.at[tsl])

        @pl.when(c != 0)
        def _():
            pltpu.sync_copy(z_hbm, acc_sh.at[tsl])

        plsc.subcore_barrier()
        base = l * NCH
        pltpu.async_copy(hs_hbm.at[idx_s.at[base]], rows_v.at[0], sem0)
        pltpu.async_copy(hs_hbm.at[idx_s.at[base + 1]], rows_v.at[1], sem1)

        def body(i, carry):
            j0 = base + 2 * i
            j1 = j0 + 1
            pltpu.make_async_copy(hs_hbm.at[idx_s.at[j0]],
                                  rows_v.at[0], sem0).wait()
            pltpu.sync_copy(rows_v.at[0], acc_sh.at[idx_r.at[j0]], add=True)
            pltpu.async_copy(hs_hbm.at[idx_s.at[j0 + 2]], rows_v.at[0], sem0)
            pltpu.make_async_copy(hs_hbm.at[idx_s.at[j1]],
                                  rows_v.at[1], sem1).wait()
            pltpu.sync_copy(rows_v.at[1], acc_sh.at[idx_r.at[j1]], add=True)
            pltpu.async_copy(hs_hbm.at[idx_s.at[j1 + 2]], rows_v.at[1], sem1)
            return carry

        lax.fori_loop(0, NCH // 2 - 1, body, 0)
        jt = base + NCH - 2
        pltpu.make_async_copy(hs_hbm.at[idx_s.at[jt]], rows_v.at[0], sem0).wait()
        pltpu.sync_copy(rows_v.at[0], acc_sh.at[idx_r.at[jt]], add=True)
        pltpu.make_async_copy(hs_hbm.at[idx_s.at[jt + 1]],
                              rows_v.at[1], sem1).wait()
        pltpu.sync_copy(rows_v.at[1], acc_sh.at[idx_r.at[jt + 1]], add=True)
        plsc.subcore_barrier()
        pltpu.sync_copy(acc_sh.at[tsl], parts_out.at[c, l, tsl])
        plsc.subcore_barrier()


def _scat_call(hs_flat, s_w, r_w, zeros2):
    f = pl.kernel(
        _scat_body,
        out_type=jax.ShapeDtypeStruct((NC, NL, NPAD, OUT), jnp.float32),
        mesh=_MESH,
        scratch_types=[
            pltpu.VMEM((TOT_CH, CH), jnp.int32),
            pltpu.VMEM((TOT_CH, CH), jnp.int32),
            pltpu.VMEM((2, CH, OUT), jnp.float32),
            pltpu.VMEM_SHARED((NPAD, OUT), jnp.float32),
            pltpu.SemaphoreType.DMA,
            pltpu.SemaphoreType.DMA,
        ],
    )
    return f(hs_flat, s_w, r_w, zeros2)


_BLK = 256


def _proj_body(n_ref, w_ref, b_ref, h_ref, o_ref):
    h = jnp.dot(n_ref[...], w_ref[0], preferred_element_type=jnp.float32)
    h = h + b_ref[0, 0][None, :]
    deg = h_ref[0, 0] + h_ref[0, 1] + 1.0
    o_ref[...] = (h * lax.rsqrt(deg)[:, None])[None]


def _proj_call(nodes_p, w_stack, b_stack, hist_s):
    return pl.pallas_call(
        _proj_body,
        grid=(NPAD // _BLK, NL),
        in_specs=[
            pl.BlockSpec((_BLK, D), lambda i, l: (i, 0)),
            pl.BlockSpec((1, D, OUT), lambda i, l: (l, 0, 0)),
            pl.BlockSpec((1, 1, OUT), lambda i, l: (l, 0, 0)),
            pl.BlockSpec((1, NC, _BLK), lambda i, l: (l, 0, i)),
        ],
        out_specs=pl.BlockSpec((1, _BLK, OUT), lambda i, l: (l, i, 0)),
        out_shape=jax.ShapeDtypeStruct((NL, NPAD, OUT), jnp.float32),
    )(nodes_p, w_stack, b_stack, hist_s)


def _final_body(p_ref, h_ref, wf_ref, bf_ref, o_ref):
    acc = jnp.zeros((_BLK, OUT), jnp.float32)
    for l in range(NL):
        p = p_ref[0, l] + p_ref[1, l]
        deg = h_ref[l, 0] + h_ref[l, 1] + 1.0
        xl = p * lax.rsqrt(deg)[:, None]
        acc = acc + jnp.dot(xl, wf_ref[l], preferred_element_type=jnp.float32)
    o_ref[...] = jnp.maximum(acc + bf_ref[0][None, :], 0.0)


def _final_call(parts, hist_r, wf_stack, bf_row):
    return pl.pallas_call(
        _final_body,
        grid=(NPAD // _BLK,),
        in_specs=[
            pl.BlockSpec((NC, NL, _BLK, OUT), lambda i: (0, 0, i, 0)),
            pl.BlockSpec((NL, NC, _BLK), lambda i: (0, 0, i)),
            pl.BlockSpec((NL, OUT, OUT), lambda i: (0, 0, 0)),
            pl.BlockSpec((1, OUT), lambda i: (0, 0)),
        ],
        out_specs=pl.BlockSpec((_BLK, OUT), lambda i: (i, 0)),
        out_shape=jax.ShapeDtypeStruct((NPAD, OUT), jnp.float32),
    )(parts, hist_r, wf_stack, bf_row)


def kernel(nodes, senders, receivers, grid_senders, grid_receivers,
           active_senders, active_receivers, passive_senders, passive_receivers,
           W0, b0, W1, b1, W2, b2, W3, b3, W4, b4, Wf, bf):
    s_list = [senders, receivers, grid_senders, active_senders, passive_senders]
    r_list = [receivers, senders, grid_receivers, active_receivers,
              passive_receivers]
    S = jnp.stack([x.astype(jnp.int32) for x in s_list])
    R = jnp.stack([x.astype(jnp.int32) for x in r_list])
    npad_e = EPAD - E
    trash = NPAD - N
    pad_idx = N + (jnp.arange(npad_e, dtype=jnp.int32) % trash)
    pad_blk = jnp.broadcast_to(pad_idx, (NL, npad_e))
    Sp = jnp.concatenate([S, pad_blk], axis=1)
    Rp = jnp.concatenate([R, pad_blk], axis=1)
    offs = (jnp.arange(NL, dtype=jnp.int32) * NPAD)[:, None]
    So = Sp + offs
    Rf = Rp + offs

    def to_w(a):
        return a.reshape(NL, NW, NCH, CH).transpose(1, 0, 2, 3).reshape(
            NW, TOT_CH, CH)

    S_w, R_w, Rf_w = to_w(So), to_w(Rp), to_w(Rf)

    nodes_p = jnp.pad(nodes, ((0, NPAD - N), (0, 0)))
    w_stack = jnp.stack([W0, W1, W2, W3, W4])
    b_stack = jnp.stack([b0, b1, b2, b3, b4]).reshape(NL, 1, OUT)
    wf_stack = Wf.reshape(NL, OUT, OUT)
    bf_row = bf.reshape(1, OUT)
    zeros1 = jnp.zeros((ROWS_PER_TILE,), jnp.float32)
    zeros2 = jnp.zeros((ROWS_PER_TILE, OUT), jnp.float32)

    hist_s, hist_r = _hist_call(S_w, Rf_w, zeros1)
    hs = _proj_call(nodes_p, w_stack, b_stack, hist_s)
    parts = _scat_call(hs.reshape(NL * NPAD, OUT), S_w, R_w, zeros2)
    out = _final_call(parts, hist_r, wf_stack, bf_row)
    return out[:N]


# speedup vs baseline: 5.9309x; 5.9309x over previous
import jax
import jax.numpy as jnp
from jax import lax
from jax.experimental import pallas as pl
from jax.experimental.pallas import tpu as pltpu
from jax.experimental.pallas import tpu_sc as plsc

N = 10000
D = 256
OUT = 128
HALF = OUT // 2
E = 160000

NPAD = 10240
RPT = NPAD // 16
NC, NS = 2, 16
CH = 128
NCHL = 80
EPAD = NS * NCHL * CH
NL = 5
TOT_CH = NL * NCHL

_MESH = plsc.VectorSubcoreMesh(
    core_axis_name="c", subcore_axis_name="s", num_cores=NC, num_subcores=NS)


def _hist_body(s_hbm, r_hbm, z_hbm, hs_out, hr_out, idx_v, ones_v, hs_sh, hr_sh):
    c = lax.axis_index("c")
    s = lax.axis_index("s")
    one = jnp.ones((16,), jnp.float32)
    for i in range(8):
        ones_v[pl.ds(i * 16, 16)] = one
    for k in range(NL):
        pltpu.sync_copy(z_hbm, hs_sh.at[pl.ds(k * NPAD + s * RPT, RPT)])
        pltpu.sync_copy(z_hbm, hr_sh.at[pl.ds(k * NPAD + s * RPT, RPT)])
    plsc.subcore_barrier()

    pltpu.sync_copy(s_hbm.at[s], idx_v)
    for k in range(NL):
        def sbody(j, carry):
            pltpu.sync_copy(ones_v,
                            hs_sh.at[idx_v.at[k * NCHL + c * (NCHL // 2) + j]],
                            add=True)
            return carry
        lax.fori_loop(0, NCHL // 2, sbody, 0)

    pltpu.sync_copy(r_hbm.at[s], idx_v)
    for k in range(NL):
        def rbody(j, carry):
            pltpu.sync_copy(ones_v,
                            hr_sh.at[idx_v.at[k * NCHL + c * (NCHL // 2) + j]],
                            add=True)
            return carry
        lax.fori_loop(0, NCHL // 2, rbody, 0)
    plsc.subcore_barrier()
    for k in range(NL):
        sl = pl.ds(k * NPAD + s * RPT, RPT)
        osl = pl.ds(s * RPT, RPT)
        pltpu.sync_copy(hs_sh.at[sl], hs_out.at[k, c, osl])
        pltpu.sync_copy(hr_sh.at[sl], hr_out.at[k, c, osl])


def _hist_call(s_w, r_w, zeros1):
    f = pl.kernel(
        _hist_body,
        out_type=(jax.ShapeDtypeStruct((NL, NC, NPAD), jnp.float32),
                  jax.ShapeDtypeStruct((NL, NC, NPAD), jnp.float32)),
        mesh=_MESH,
        scratch_types=[
            pltpu.VMEM((TOT_CH, CH), jnp.int32),
            pltpu.VMEM((CH,), jnp.float32),
            pltpu.VMEM_SHARED((NL * NPAD,), jnp.float32),
            pltpu.VMEM_SHARED((NL * NPAD,), jnp.float32),
        ],
    )
    return f(s_w, r_w, zeros1)


def _scat_body(hs_hbm, s0_hbm, s1_hbm, r_hbm, parts_out,
               idx_s, idx_r, rows_v, acc_sh, sem0, sem1):
    c = lax.axis_index("c")
    s = lax.axis_index("s")
    tsl = pl.ds(s * RPT, RPT)
    for l in range(NL):
        @pl.when(c == 0)
        def _():
            pltpu.sync_copy(hs_hbm.at[pl.ds(l * NPAD + s * RPT, RPT)],
                            acc_sh.at[tsl])
            pltpu.sync_copy(s0_hbm.at[s, pl.ds(l * NCHL, NCHL)], idx_s)

        @pl.when(c != 0)
        def _():
            pltpu.sync_copy(hs_hbm.at[pl.ds((NL + l) * NPAD + s * RPT, RPT)],
                            acc_sh.at[tsl])
            pltpu.sync_copy(s1_hbm.at[s, pl.ds(l * NCHL, NCHL)], idx_s)

        pltpu.sync_copy(r_hbm.at[s, pl.ds(l * NCHL, NCHL)], idx_r)
        plsc.subcore_barrier()
        pltpu.async_copy(hs_hbm.at[idx_s.at[0]], rows_v.at[0], sem0)
        pltpu.async_copy(hs_hbm.at[idx_s.at[1]], rows_v.at[1], sem1)

        def body(i, carry):
            j0 = 2 * i
            j1 = j0 + 1
            pltpu.make_async_copy(hs_hbm.at[idx_s.at[j0]],
                                  rows_v.at[0], sem0).wait()
            pltpu.sync_copy(rows_v.at[0], acc_sh.at[idx_r.at[j0]], add=True)
            pltpu.async_copy(hs_hbm.at[idx_s.at[j0 + 2]], rows_v.at[0], sem0)
            pltpu.make_async_copy(hs_hbm.at[idx_s.at[j1]],
                                  rows_v.at[1], sem1).wait()
            pltpu.sync_copy(rows_v.at[1], acc_sh.at[idx_r.at[j1]], add=True)
            pltpu.async_copy(hs_hbm.at[idx_s.at[j1 + 2]], rows_v.at[1], sem1)
            return carry

        lax.fori_loop(0, NCHL // 2 - 1, body, 0)
        jt = NCHL - 2
        pltpu.make_async_copy(hs_hbm.at[idx_s.at[jt]], rows_v.at[0], sem0).wait()
        pltpu.sync_copy(rows_v.at[0], acc_sh.at[idx_r.at[jt]], add=True)
        pltpu.make_async_copy(hs_hbm.at[idx_s.at[jt + 1]],
                              rows_v.at[1], sem1).wait()
        pltpu.sync_copy(rows_v.at[1], acc_sh.at[idx_r.at[jt + 1]], add=True)
        plsc.subcore_barrier()
        pltpu.sync_copy(acc_sh.at[tsl], parts_out.at[c, l, tsl])
        plsc.subcore_barrier()


def _scat_call(hs_flat, s_w0, s_w1, r_w):
    f = pl.kernel(
        _scat_body,
        out_type=jax.ShapeDtypeStruct((NC, NL, NPAD, HALF), jnp.float32),
        mesh=_MESH,
        compiler_params=pltpu.CompilerParams(use_tc_tiling_on_sc=False),
        scratch_types=[
            pltpu.VMEM((NCHL, CH), jnp.int32),
            pltpu.VMEM((NCHL, CH), jnp.int32),
            pltpu.VMEM((2, CH, HALF), jnp.float32),
            pltpu.VMEM_SHARED((NPAD, HALF), jnp.float32),
            pltpu.SemaphoreType.DMA,
            pltpu.SemaphoreType.DMA,
        ],
    )
    return f(hs_flat, s_w0, s_w1, r_w)


_BLK = 256


def _proj_body(n_ref, w_ref, b_ref, h_ref, o_ref):
    h = jnp.dot(n_ref[...], w_ref[0], preferred_element_type=jnp.float32)
    h = h + b_ref[0, 0][None, :]
    deg = h_ref[0, 0] + h_ref[0, 1] + 1.0
    h = h * lax.rsqrt(deg)[:, None]
    o_ref[0, 0] = h[:, :HALF]
    o_ref[1, 0] = h[:, HALF:]


def _proj_call(nodes_p, w_stack, b_stack, hist_s):
    return pl.pallas_call(
        _proj_body,
        grid=(NPAD // _BLK, NL),
        in_specs=[
            pl.BlockSpec((_BLK, D), lambda i, l: (i, 0)),
            pl.BlockSpec((1, D, OUT), lambda i, l: (l, 0, 0)),
            pl.BlockSpec((1, 1, OUT), lambda i, l: (l, 0, 0)),
            pl.BlockSpec((1, NC, _BLK), lambda i, l: (l, 0, i)),
        ],
        out_specs=pl.BlockSpec((NC, 1, _BLK, HALF), lambda i, l: (0, l, i, 0)),
        out_shape=jax.ShapeDtypeStruct((NC, NL, NPAD, HALF), jnp.float32),
    )(nodes_p, w_stack, b_stack, hist_s)


def _final_body(p_ref, h_ref, wf_ref, bf_ref, o_ref):
    acc = jnp.zeros((_BLK, OUT), jnp.float32)
    for l in range(NL):
        p = jnp.concatenate([p_ref[0, l], p_ref[1, l]], axis=1)
        deg = h_ref[l, 0] + h_ref[l, 1] + 1.0
        xl = p * lax.rsqrt(deg)[:, None]
        acc = acc + jnp.dot(xl, wf_ref[l], preferred_element_type=jnp.float32)
    o_ref[...] = jnp.maximum(acc + bf_ref[0][None, :], 0.0)


def _final_call(parts, hist_r, wf_stack, bf_row):
    return pl.pallas_call(
        _final_body,
        grid=(NPAD // _BLK,),
        in_specs=[
            pl.BlockSpec((NC, NL, _BLK, HALF), lambda i: (0, 0, i, 0)),
            pl.BlockSpec((NL, NC, _BLK), lambda i: (0, 0, i)),
            pl.BlockSpec((NL, OUT, OUT), lambda i: (0, 0, 0)),
            pl.BlockSpec((1, OUT), lambda i: (0, 0)),
        ],
        out_specs=pl.BlockSpec((_BLK, OUT), lambda i: (i, 0)),
        out_shape=jax.ShapeDtypeStruct((NPAD, OUT), jnp.float32),
    )(parts, hist_r, wf_stack, bf_row)


def kernel(nodes, senders, receivers, grid_senders, grid_receivers,
           active_senders, active_receivers, passive_senders, passive_receivers,
           W0, b0, W1, b1, W2, b2, W3, b3, W4, b4, Wf, bf):
    s_list = [senders, receivers, grid_senders, active_senders, passive_senders]
    r_list = [receivers, senders, grid_receivers, active_receivers,
              passive_receivers]
    S = jnp.stack([x.astype(jnp.int32) for x in s_list])
    R = jnp.stack([x.astype(jnp.int32) for x in r_list])
    npad_e = EPAD - E
    trash = NPAD - N
    pad_idx = N + (jnp.arange(npad_e, dtype=jnp.int32) % trash)
    pad_blk = jnp.broadcast_to(pad_idx, (NL, npad_e))
    Sp = jnp.concatenate([S, pad_blk], axis=1)
    Rp = jnp.concatenate([R, pad_blk], axis=1)
    offs = (jnp.arange(NL, dtype=jnp.int32) * NPAD)[:, None]
    So = Sp + offs
    Rf = Rp + offs

    def to_w(a):
        return a.reshape(NL, NS, NCHL, CH).transpose(1, 0, 2, 3).reshape(
            NS, TOT_CH, CH)

    S_w0 = to_w(So)
    S_w1 = S_w0 + NL * NPAD
    R_w, Rf_w = to_w(Rp), to_w(Rf)

    nodes_p = jnp.pad(nodes, ((0, NPAD - N), (0, 0)))
    w_stack = jnp.stack([W0, W1, W2, W3, W4])
    b_stack = jnp.stack([b0, b1, b2, b3, b4]).reshape(NL, 1, OUT)
    wf_stack = Wf.reshape(NL, OUT, OUT)
    bf_row = bf.reshape(1, OUT)
    zeros1 = jnp.zeros((RPT,), jnp.float32)

    hist_s, hist_r = _hist_call(S_w0, Rf_w, zeros1)
    hs = _proj_call(nodes_p, w_stack, b_stack, hist_s)
    parts = _scat_call(hs.reshape(NC * NL * NPAD, HALF), S_w0, S_w1, R_w)
    out = _final_call(parts, hist_r, wf_stack, bf_row)
    return out[:N]
